# Initial kernel scaffold; baseline (speedup 1.0000x reference)
import jax
import jax.numpy as jnp
from jax.experimental import pallas as pl


def kernel(pred_coords, target_coords, target_bitmap):
    raise NotImplementedError("write your pallas kernel here")


# speedup vs baseline: 876.7155x; 876.7155x over previous
import functools

import jax
import jax.numpy as jnp
import numpy as np
from jax import lax
from jax.experimental import pallas as pl
from jax.experimental.pallas import tpu as pltpu
from jax.experimental.pallas import tpu_sc as plsc

NC, NS = 2, 16
NW = NC * NS
BATCH = 2048
NPTS = 128
NSEG = NPTS - 1
HW = 28
NPIX = HW * HW
LANES = 16
NCHUNK = BATCH // LANES
CPW = NCHUNK // NW

_EPS = np.float32(1e-7)
_PSET = np.float32(np.float32(1.0) - _EPS)
_A = np.float32(np.log(_EPS))
_B = np.float32(np.log(_PSET))
_A2 = np.float32(np.log(np.float32(np.float32(1.0) - _PSET)))


def _sc_body(pxr, pyr, txr, tyr, bmr, out_hbm,
             pxv, pyv, txv, tyv, bmv, canvas, outv):
    wid = lax.axis_index("c") * NS + lax.axis_index("s")

    lane = lax.iota(jnp.int32, LANES)
    zeros = jnp.zeros((LANES,), jnp.float32)
    ones = jnp.ones((LANES,), jnp.float32)
    izeros = jnp.zeros((LANES,), jnp.int32)

    def zb(p, _):
        canvas[pl.ds(p * LANES, LANES)] = zeros
        return 0
    lax.fori_loop(0, NPIX, zb, 0)

    n_acc = zeros
    t_acc = zeros
    ta_acc = zeros
    mse_acc = zeros

    for j in range(CPW):
        c = wid * CPW + j
        pltpu.sync_copy(pxr.at[c], pxv)
        pltpu.sync_copy(pyr.at[c], pyv)
        pltpu.sync_copy(txr.at[c], txv)
        pltpu.sync_copy(tyr.at[c], tyv)
        pltpu.sync_copy(bmr.at[c], bmv)

        def seg_body(k, _):
            x0f = pxv[pl.ds(k * LANES, LANES)]
            y0f = pyv[pl.ds(k * LANES, LANES)]
            x1f = pxv[pl.ds((k + 1) * LANES, LANES)]
            y1f = pyv[pl.ds((k + 1) * LANES, LANES)]
            s = jnp.float32(HW - 1)
            x0 = (x0f * s).astype(jnp.int32)
            y0 = (y0f * s).astype(jnp.int32)
            x1 = (x1f * s).astype(jnp.int32)
            y1 = (y1f * s).astype(jnp.int32)

            steep = jnp.abs(y1 - y0) > jnp.abs(x1 - x0)
            ax0 = jnp.where(steep, y0, x0)
            ay0 = jnp.where(steep, x0, y0)
            ax1 = jnp.where(steep, y1, x1)
            ay1 = jnp.where(steep, x1, y1)
            swap = ax0 > ax1
            bx0 = jnp.where(swap, ax1, ax0)
            bx1 = jnp.where(swap, ax0, ax1)
            by0 = jnp.where(swap, ay1, ay0)
            by1 = jnp.where(swap, ay0, ay1)
            dx = bx1 - bx0
            dy = jnp.abs(by1 - by0)
            ystep = jnp.where(by0 < by1, jnp.int32(1), jnp.int32(-1))
            den = jnp.maximum(dx, 1)

            def i_body(i, st):
                rem, xx, yy = st
                m = i <= dx
                rr = jnp.where(steep, xx, yy)
                cc = jnp.where(steep, yy, xx)
                plsc.store_scatter(
                    canvas, [(rr * HW + cc) * LANES + lane], ones, mask=m)
                rem = rem + dy
                carry = rem >= den
                rem = rem - jnp.where(carry, den, 0)
                yy = yy + jnp.where(carry, ystep, 0)
                xx = xx + 1
                return (rem, xx, yy)

            lax.fori_loop(0, HW, i_body, (izeros, bx0, by0))
            return 0

        lax.fori_loop(0, NSEG, seg_body, 0)

        def red_body(p, accs):
            na, ta, taa = accs
            cv = canvas[pl.ds(p * LANES, LANES)]
            canvas[pl.ds(p * LANES, LANES)] = zeros
            t = bmv[pl.ds(p * LANES, LANES)]
            return (na + cv, ta + cv * t, taa + t)

        n_acc, t_acc, ta_acc = lax.fori_loop(
            0, NPIX, red_body, (n_acc, t_acc, ta_acc))

        def mse_body(k, acc):
            o = k * LANES
            d0 = pxv[pl.ds(o, LANES)] - txv[pl.ds(o, LANES)]
            d1 = pyv[pl.ds(o, LANES)] - tyv[pl.ds(o, LANES)]
            return acc + d0 * d0 + d1 * d1

        mse_acc = lax.fori_loop(0, NPTS, mse_body, mse_acc)

    outv[pl.ds(0, LANES)] = n_acc
    outv[pl.ds(LANES, LANES)] = t_acc
    outv[pl.ds(2 * LANES, LANES)] = ta_acc
    outv[pl.ds(3 * LANES, LANES)] = mse_acc
    pltpu.sync_copy(outv, out_hbm.at[wid])


@functools.partial(jax.jit, static_argnames=())
def kernel(pred_coords, target_coords, target_bitmap):
    def chunked(a):
        return a.reshape(NCHUNK, LANES, NPTS).transpose(0, 2, 1).reshape(
            NCHUNK, NPTS * LANES)

    pxr = chunked(pred_coords[:, :, 0])
    pyr = chunked(pred_coords[:, :, 1])
    txr = chunked(target_coords[:, :, 0])
    tyr = chunked(target_coords[:, :, 1])
    bmr = target_bitmap.reshape(NCHUNK, LANES, NPIX).transpose(0, 2, 1).reshape(
        NCHUNK, NPIX * LANES)

    mesh = plsc.VectorSubcoreMesh(
        core_axis_name="c", subcore_axis_name="s",
        num_cores=NC, num_subcores=NS)

    run = pl.kernel(
        _sc_body,
        out_type=jax.ShapeDtypeStruct((NW, 4 * LANES), jnp.float32),
        mesh=mesh,
        compiler_params=pltpu.CompilerParams(needs_layout_passes=False),
        scratch_types=[
            pltpu.VMEM((NPTS * LANES,), jnp.float32),
            pltpu.VMEM((NPTS * LANES,), jnp.float32),
            pltpu.VMEM((NPTS * LANES,), jnp.float32),
            pltpu.VMEM((NPTS * LANES,), jnp.float32),
            pltpu.VMEM((NPIX * LANES,), jnp.float32),
            pltpu.VMEM((NPIX * LANES,), jnp.float32),
            pltpu.VMEM((4 * LANES,), jnp.float32),
        ],
    )

    parts = run(pxr, pyr, txr, tyr, bmr)
    parts = parts.reshape(NW, 4, LANES).sum(axis=(0, 2))
    n_set, t_set, t_all, sse = parts[0], parts[1], parts[2], parts[3]

    n_pix = np.float32(BATCH * NPIX)
    n_coord = np.float32(BATCH * NPTS * 2)
    coord_loss = sse / n_coord
    bce_sum = ((-_B) * n_pix - (_A - _B) * t_all
               + (_B - _A2) * n_set + (_A2 + _A - 2.0 * _B) * t_set)
    raster_loss = bce_sum / n_pix
    total_loss = (np.float32(1.0) * coord_loss
                  + np.float32(0.5) * raster_loss)
    return (coord_loss, raster_loss, total_loss)
